# Initial kernel scaffold; baseline (speedup 1.0000x reference)
import jax
import jax.numpy as jnp
from jax.experimental import pallas as pl


def kernel(x, weight):
    raise NotImplementedError("write your pallas kernel here")


# speedup vs baseline: 1.3325x; 1.3325x over previous
import functools

import jax
import jax.numpy as jnp
from jax import lax
from jax.experimental import pallas as pl
from jax.experimental.pallas import tpu as pltpu

TOKENS = 16384
N_EMBD = 2048
N_EXPERTS = 8
TOPK = 2
BLOCK = 1024


def _gate_block(x_ref, w_ref, scores_ref, weights_ref, indices_ref):
    scores = jnp.dot(x_ref[...], w_ref[...], preferred_element_type=jnp.float32)
    sig = jax.nn.sigmoid(scores)
    scores_ref[...] = sig

    col = lax.broadcasted_iota(jnp.int32, sig.shape, 1)
    m1 = jnp.max(sig, axis=1, keepdims=True)
    i1 = jnp.min(jnp.where(sig == m1, col, N_EXPERTS), axis=1, keepdims=True)
    rest = jnp.where(col == i1, -1.0, sig)
    m2 = jnp.max(rest, axis=1, keepdims=True)
    i2 = jnp.min(jnp.where(rest == m2, col, N_EXPERTS), axis=1, keepdims=True)

    denom = m1 + m2 + 1e-6
    weights_ref[...] = jnp.concatenate([m1 / denom, m2 / denom], axis=1)
    indices_ref[...] = jnp.concatenate([i1, i2], axis=1)


def kernel(x, weight, interpret=False):
    n_tokens = x.shape[0]
    grid = (n_tokens // BLOCK,)
    out = pl.pallas_call(
        _gate_block,
        grid=grid,
        in_specs=[
            pl.BlockSpec((BLOCK, N_EMBD), lambda i: (i, 0)),
            pl.BlockSpec((N_EMBD, N_EXPERTS), lambda i: (0, 0)),
        ],
        out_specs=[
            pl.BlockSpec((BLOCK, N_EXPERTS), lambda i: (i, 0)),
            pl.BlockSpec((BLOCK, TOPK), lambda i: (i, 0)),
            pl.BlockSpec((BLOCK, TOPK), lambda i: (i, 0)),
        ],
        out_shape=[
            jax.ShapeDtypeStruct((n_tokens, N_EXPERTS), jnp.float32),
            jax.ShapeDtypeStruct((n_tokens, TOPK), jnp.float32),
            jax.ShapeDtypeStruct((n_tokens, TOPK), jnp.int32),
        ],
        compiler_params=pltpu.CompilerParams(
            dimension_semantics=("arbitrary",),
        ),
        interpret=interpret,
    )(x, weight)
    return tuple(out)


# speedup vs baseline: 1.3516x; 1.0143x over previous
import functools

import jax
import jax.numpy as jnp
from jax import lax
from jax.experimental import pallas as pl
from jax.experimental.pallas import tpu as pltpu

TOKENS = 16384
N_EMBD = 2048
N_EXPERTS = 8
TOPK = 2
BLOCK = 2048


def _gate_block(x_ref, w_ref, scores_ref, weights_ref, indices_ref):
    scores = jnp.dot(x_ref[...], w_ref[...], preferred_element_type=jnp.float32)
    sig = jax.nn.sigmoid(scores)
    scores_ref[...] = sig

    col = lax.broadcasted_iota(jnp.int32, sig.shape, 1)
    m1 = jnp.max(sig, axis=1, keepdims=True)
    i1 = jnp.min(jnp.where(sig == m1, col, N_EXPERTS), axis=1, keepdims=True)
    rest = jnp.where(col == i1, -1.0, sig)
    m2 = jnp.max(rest, axis=1, keepdims=True)
    i2 = jnp.min(jnp.where(rest == m2, col, N_EXPERTS), axis=1, keepdims=True)

    denom = m1 + m2 + 1e-6
    weights_ref[...] = jnp.concatenate([m1 / denom, m2 / denom], axis=1)
    indices_ref[...] = jnp.concatenate([i1, i2], axis=1)


def kernel(x, weight, interpret=False):
    n_tokens = x.shape[0]
    grid = (n_tokens // BLOCK,)
    out = pl.pallas_call(
        _gate_block,
        grid=grid,
        in_specs=[
            pl.BlockSpec((BLOCK, N_EMBD), lambda i: (i, 0)),
            pl.BlockSpec((N_EMBD, N_EXPERTS), lambda i: (0, 0)),
        ],
        out_specs=[
            pl.BlockSpec((BLOCK, N_EXPERTS), lambda i: (i, 0)),
            pl.BlockSpec((BLOCK, TOPK), lambda i: (i, 0)),
            pl.BlockSpec((BLOCK, TOPK), lambda i: (i, 0)),
        ],
        out_shape=[
            jax.ShapeDtypeStruct((n_tokens, N_EXPERTS), jnp.float32),
            jax.ShapeDtypeStruct((n_tokens, TOPK), jnp.float32),
            jax.ShapeDtypeStruct((n_tokens, TOPK), jnp.int32),
        ],
        compiler_params=pltpu.CompilerParams(
            dimension_semantics=("arbitrary",),
        ),
        interpret=interpret,
    )(x, weight)
    return tuple(out)
